# baseline (speedup 1.0000x reference)
import functools

import jax
import jax.numpy as jnp
from jax.experimental import pallas as pl
from jax.experimental.pallas import tpu as pltpu

T = 2048
HID = 2048
E = 8
DFF = 768
TOPK = 2
EPS = 1e-6

DB = 256
TB2 = 512


def _rms_x2(x, ln2):
    var = jnp.mean(x * x, axis=-1, keepdims=True)
    return x * jax.lax.rsqrt(var + EPS) * ln2


def _combine_weights(x2, wg):
    logits = jax.lax.dot_general(x2, wg, (((1,), (1,)), ((), ())),
                                 preferred_element_type=jnp.float32)
    probs = jax.nn.softmax(logits, axis=-1)
    iota = jax.lax.broadcasted_iota(jnp.int32, probs.shape, 1)
    m1 = jnp.max(probs, axis=-1, keepdims=True)
    i1 = jnp.min(jnp.where(probs == m1, iota, E), axis=-1, keepdims=True)
    probs_m = jnp.where(iota == i1, -jnp.inf, probs)
    m2 = jnp.max(probs_m, axis=-1, keepdims=True)
    i2 = jnp.min(jnp.where(probs_m == m2, iota, E), axis=-1, keepdims=True)
    sel1 = (iota == i1).astype(jnp.float32)
    sel2 = (iota == i2).astype(jnp.float32)
    return (sel1 * m1 + sel2 * m2) / (m1 + m2)


def _moe_body(hid_ref, ln2_ref, wg_ref, w1_ref, w2_ref, out_ref,
              x2b_ref, cw_ref):
    e = pl.program_id(0)
    d = pl.program_id(1)

    @pl.when((e == 0) & (d == 0))
    def _setup():
        x2 = _rms_x2(hid_ref[...], ln2_ref[...])
        x2b_ref[...] = x2.astype(jnp.bfloat16)
        cw_ref[...] = _combine_weights(x2, wg_ref[...])
        out_ref[...] = jnp.zeros_like(out_ref)

    x2b = x2b_ref[...]
    h = jax.lax.dot_general(x2b, w1_ref[0].astype(jnp.bfloat16),
                            (((1,), (1,)), ((), ())),
                            preferred_element_type=jnp.float32)
    h = (h * jax.nn.sigmoid(h)).astype(jnp.bfloat16)
    y = jax.lax.dot_general(h, w2_ref[0].astype(jnp.bfloat16),
                            (((1,), (1,)), ((), ())),
                            preferred_element_type=jnp.float32)
    cw = cw_ref[...]
    iota = jax.lax.broadcasted_iota(jnp.int32, cw.shape, 1)
    ce = jnp.sum(jnp.where(iota == e, cw, 0.0), axis=1, keepdims=True)
    out_ref[...] += y * ce


def _shared_body(hid_ref, routed_ref, ln2_ref, wse_ref, wsd_ref, out_ref):
    x = hid_ref[...]
    x2 = _rms_x2(x, ln2_ref[...])
    gu = jax.lax.dot_general(x2.astype(jnp.bfloat16),
                             wse_ref[...].astype(jnp.bfloat16),
                             (((1,), (1,)), ((), ())),
                             preferred_element_type=jnp.float32)
    gate = gu[:, :DFF]
    up = gu[:, DFF:]
    act = (gate * jax.nn.sigmoid(gate) * up).astype(jnp.bfloat16)
    shared = jax.lax.dot_general(act, wsd_ref[...].astype(jnp.bfloat16),
                                 (((1,), (1,)), ((), ())),
                                 preferred_element_type=jnp.float32)
    out_ref[...] = x + routed_ref[...] + shared


def kernel(hidden_states, positions, kv_cache, attn_metadata, ln1_w, ln2_w,
           Wq, Wkv, Wo, Wg, w1, w2, Wse, Wsd):
    ln2 = ln2_w.reshape(1, HID)

    routed = pl.pallas_call(
        _moe_body,
        grid=(E, DFF // DB),
        in_specs=[
            pl.BlockSpec((T, HID), lambda e, d: (0, 0)),
            pl.BlockSpec((1, HID), lambda e, d: (0, 0)),
            pl.BlockSpec((E, HID), lambda e, d: (0, 0)),
            pl.BlockSpec((1, DB, HID), lambda e, d: (e, d, 0)),
            pl.BlockSpec((1, HID, DB), lambda e, d: (e, 0, d)),
        ],
        out_specs=pl.BlockSpec((T, HID), lambda e, d: (0, 0)),
        out_shape=jax.ShapeDtypeStruct((T, HID), jnp.float32),
        scratch_shapes=[
            pltpu.VMEM((T, HID), jnp.bfloat16),
            pltpu.VMEM((T, E), jnp.float32),
        ],
        compiler_params=pltpu.CompilerParams(
            dimension_semantics=("arbitrary", "arbitrary")),
    )(hidden_states, ln2, Wg, w1, w2)

    out = pl.pallas_call(
        _shared_body,
        grid=(T // TB2,),
        in_specs=[
            pl.BlockSpec((TB2, HID), lambda t: (t, 0)),
            pl.BlockSpec((TB2, HID), lambda t: (t, 0)),
            pl.BlockSpec((1, HID), lambda t: (0, 0)),
            pl.BlockSpec((2 * DFF, HID), lambda t: (0, 0)),
            pl.BlockSpec((HID, DFF), lambda t: (0, 0)),
        ],
        out_specs=pl.BlockSpec((TB2, HID), lambda t: (t, 0)),
        out_shape=jax.ShapeDtypeStruct((T, HID), jnp.float32),
    )(hidden_states, routed, ln2, Wse, Wsd)

    return out
